# baseline (speedup 1.0000x reference)
import functools

import jax
import jax.numpy as jnp
from jax import lax
from jax.experimental import pallas as pl
from jax.experimental.pallas import tpu as pltpu
from jax.experimental.pallas import tpu_sc as plsc

N_NODES = 10000
N_EDGES = 320000
D_FEAT = 128

NC = 2
NS = 16
NW = NC * NS
L = 16

CHUNK = 112
NCHUNK = 3 + 6 * (-(-(-(-N_EDGES // (NW * CHUNK)) - 3) // 6))
E_PAD = NW * NCHUNK * CHUNK
ROWS_PER_TILE = (N_NODES // NS) // 8 * 8
ROWS_REM = N_NODES - NS * ROWS_PER_TILE

_mesh = plsc.VectorSubcoreMesh(
    core_axis_name="c", subcore_axis_name="s", num_cores=NC, num_subcores=NS
)


@functools.partial(
    pl.kernel,
    out_type=jax.ShapeDtypeStruct((NC, N_NODES, D_FEAT), jnp.float32),
    mesh=_mesh,
    scratch_types=[
        [pltpu.VMEM((CHUNK, D_FEAT), jnp.float32) for _ in range(3)],
        [pltpu.VMEM((3, CHUNK), jnp.int32) for _ in range(6)],
        pltpu.VMEM_SHARED((N_NODES, D_FEAT), jnp.float32),
        [pltpu.SemaphoreType.DMA for _ in range(3)],
        [pltpu.SemaphoreType.DMA for _ in range(3)],
        [pltpu.SemaphoreType.DMA for _ in range(6)],
    ],
)
def _sc_aggregate(ir_h, feats_h, partial_h,
                  gbufs, ibufs, acc, gsems, ssems, isems):
    gbuf0 = gbufs[0]
    cid = lax.axis_index("c")
    sid = lax.axis_index("s")
    wid = cid * NS + sid

    zero16 = jnp.zeros((L,), jnp.float32)

    def _zrow(i, carry):
        for r in range(D_FEAT // L):
            gbuf0[i, pl.ds(r * L, L)] = zero16
        return carry

    lax.fori_loop(0, CHUNK, _zrow, 0)
    base = sid * ROWS_PER_TILE
    full, rem = divmod(ROWS_PER_TILE, CHUNK)
    for k in range(full):
        pltpu.sync_copy(gbuf0, acc.at[pl.ds(base + k * CHUNK, CHUNK)])
    if rem:
        pltpu.sync_copy(gbuf0.at[pl.ds(0, rem)],
                        acc.at[pl.ds(base + full * CHUNK, rem)])

    @pl.when(sid == NS - 1)
    def _zero_tail():
        pltpu.sync_copy(gbuf0.at[pl.ds(0, ROWS_REM)],
                        acc.at[pl.ds(NS * ROWS_PER_TILE, ROWS_REM)])

    plsc.subcore_barrier()

    def _scale(b, i6, j):
        buf = gbufs[b]
        wrow = ibufs[i6]

        def _group(g, carry):
            w16 = lax.bitcast_convert_type(
                wrow[2, pl.ds(g * L, L)], jnp.float32)
            for i in range(L):
                e = g * L + i
                wi = jnp.broadcast_to(w16[i], (L,))
                for r in range(D_FEAT // L):
                    buf[e, pl.ds(r * L, L)] = buf[e, pl.ds(r * L, L)] * wi
            return carry

        lax.fori_loop(0, CHUNK // L, _group, 0)

    def _start_fetch(j, i6):
        pltpu.async_copy(ir_h.at[wid, j], ibufs[i6], isems[i6])

    def _wait_fetch(i6):
        pltpu.make_async_copy(ir_h.at[wid, 0], ibufs[i6], isems[i6]).wait()

    def _start_gather(b, i6):
        pltpu.async_copy(feats_h.at[ibufs[i6].at[0]], gbufs[b], gsems[b])

    def _wait_gather(b, i6):
        pltpu.make_async_copy(feats_h.at[ibufs[i6].at[0]], gbufs[b],
                              gsems[b]).wait()

    def _start_scatter(b, i6):
        pltpu.async_copy(gbufs[b], acc.at[ibufs[i6].at[1]], ssems[b], add=True)

    def _wait_scatter(b):
        pltpu.make_async_copy(gbufs[b], acc.at[ibufs[0].at[1]],
                              ssems[b]).wait()

    _start_fetch(0, 0)
    _start_fetch(1, 1)
    _wait_fetch(0)
    _start_gather(0, 0)

    _wait_gather(0, 0)
    _wait_fetch(1)
    _start_gather(1, 1)
    _start_fetch(2, 2)
    _scale(0, 0, 0)
    _start_scatter(0, 0)

    _wait_gather(1, 1)
    _wait_fetch(2)
    _start_gather(2, 2)
    _start_fetch(3, 3)
    _scale(1, 1, 1)
    _start_scatter(1, 1)

    _wait_gather(2, 2)
    _wait_scatter(0)
    _wait_fetch(3)
    _start_gather(0, 3)
    _start_fetch(4, 4)
    _scale(2, 2, 2)
    _start_scatter(2, 2)

    def _six(t, carry):
        for k in range(6):
            j = 6 * t + 3 + k
            b = k % 3
            i6 = (3 + k) % 6
            _wait_gather(b, i6)
            _wait_scatter((b + 1) % 3)

            @pl.when(j + 1 < NCHUNK)
            def _next_gather():
                _wait_fetch((i6 + 1) % 6)
                _start_gather((b + 1) % 3, (i6 + 1) % 6)

            @pl.when(j + 2 < NCHUNK)
            def _next_fetch():
                _start_fetch(j + 2, (i6 + 2) % 6)

            _scale(b, i6, j)
            _start_scatter(b, i6)
        return carry

    lax.fori_loop(0, (NCHUNK - 3) // 6, _six, 0)

    _wait_scatter((NCHUNK - 2) % 3)
    _wait_scatter((NCHUNK - 1) % 3)

    plsc.subcore_barrier()
    pltpu.sync_copy(acc.at[pl.ds(base, ROWS_PER_TILE)],
                    partial_h.at[cid, pl.ds(base, ROWS_PER_TILE)])

    @pl.when(sid == NS - 1)
    def _flush_tail():
        pltpu.sync_copy(
            acc.at[pl.ds(NS * ROWS_PER_TILE, ROWS_REM)],
            partial_h.at[cid, pl.ds(NS * ROWS_PER_TILE, ROWS_REM)])


def _combine_body(p_ref, o_ref):
    o_ref[...] = p_ref[0] + p_ref[1]


_ROWS_BLK = 1000


@jax.jit
def _combine(partial):
    return pl.pallas_call(
        _combine_body,
        out_shape=jax.ShapeDtypeStruct((N_NODES, D_FEAT), jnp.float32),
        grid=(N_NODES // _ROWS_BLK,),
        in_specs=[pl.BlockSpec((NC, _ROWS_BLK, D_FEAT), lambda i: (0, i, 0))],
        out_specs=pl.BlockSpec((_ROWS_BLK, D_FEAT), lambda i: (i, 0)),
    )(partial)


@jax.jit
def kernel(edge_index, edge_weight, feats):
    rows = edge_index[0].astype(jnp.int32)
    cols = edge_index[1].astype(jnp.int32)
    w = lax.bitcast_convert_type(edge_weight.astype(jnp.float32), jnp.int32)

    pad = E_PAD - N_EDGES
    rows = jnp.concatenate([rows, jnp.zeros((pad,), jnp.int32)])
    cols = jnp.concatenate([cols, jnp.zeros((pad,), jnp.int32)])
    w = jnp.concatenate([w, jnp.zeros((pad,), jnp.int32)])

    ir = jnp.stack([cols.reshape(NW, NCHUNK, CHUNK),
                    rows.reshape(NW, NCHUNK, CHUNK),
                    w.reshape(NW, NCHUNK, CHUNK)], axis=2)

    partial = _sc_aggregate(ir, feats)
    return _combine(partial)


# speedup vs baseline: 1.9148x; 1.9148x over previous
import functools

import jax
import jax.numpy as jnp
from jax import lax
from jax.experimental import pallas as pl
from jax.experimental.pallas import tpu as pltpu
from jax.experimental.pallas import tpu_sc as plsc

N_NODES = 10000
N_EDGES = 320000
D_FEAT = 128

NC = 2
NS = 16
NW = NC * NS
L = 16

CHUNK = 128
NCHUNK = -(-N_EDGES // (NW * CHUNK))
E_PAD = NW * NCHUNK * CHUNK
ROWS_PER_TILE = (N_NODES // NS) // 8 * 8
ROWS_REM = N_NODES - NS * ROWS_PER_TILE

_mesh = plsc.VectorSubcoreMesh(
    core_axis_name="c", subcore_axis_name="s", num_cores=NC, num_subcores=NS
)


@functools.partial(
    pl.kernel,
    out_type=jax.ShapeDtypeStruct((NC, N_NODES, D_FEAT), jnp.float32),
    mesh=_mesh,
    scratch_types=[
        pltpu.VMEM((NCHUNK, CHUNK), jnp.int32),
        pltpu.VMEM((NCHUNK, CHUNK), jnp.int32),
        pltpu.VMEM((NCHUNK * CHUNK,), jnp.float32),
        pltpu.VMEM((CHUNK, D_FEAT), jnp.float32),
        pltpu.VMEM_SHARED((N_NODES, D_FEAT), jnp.float32),
        pltpu.SemaphoreType.DMA,
    ],
)
def _sc_aggregate(cols_h, rows_h, w_h, feats_h, partial_h,
                  cols_v, rows_v, w_v, gbuf, acc, sem):
    cid = lax.axis_index("c")
    sid = lax.axis_index("s")
    wid = cid * NS + sid

    pltpu.sync_copy(cols_h.at[wid], cols_v)
    pltpu.sync_copy(rows_h.at[wid], rows_v)
    pltpu.sync_copy(w_h.at[wid], w_v)

    zero16 = jnp.zeros((L,), jnp.float32)

    def _zrow(i, carry):
        for r in range(D_FEAT // L):
            gbuf[i, pl.ds(r * L, L)] = zero16
        return carry

    lax.fori_loop(0, CHUNK, _zrow, 0)
    base = sid * ROWS_PER_TILE
    full, rem = divmod(ROWS_PER_TILE, CHUNK)
    for k in range(full):
        pltpu.sync_copy(gbuf, acc.at[pl.ds(base + k * CHUNK, CHUNK)])
    if rem:
        pltpu.sync_copy(gbuf.at[pl.ds(0, rem)],
                        acc.at[pl.ds(base + full * CHUNK, rem)])

    @pl.when(sid == NS - 1)
    def _zero_tail():
        pltpu.sync_copy(gbuf.at[pl.ds(0, ROWS_REM)],
                        acc.at[pl.ds(NS * ROWS_PER_TILE, ROWS_REM)])

    plsc.subcore_barrier()

    def _chunk(j, carry):
        pltpu.async_copy(feats_h.at[cols_v.at[j]], gbuf, sem).wait()

        jbase = j * CHUNK

        def _group(g, carry):
            w16 = w_v[pl.ds(jbase + g * L, L)]
            for i in range(L):
                e = g * L + i
                wi = jnp.broadcast_to(w16[i], (L,))
                for r in range(D_FEAT // L):
                    gbuf[e, pl.ds(r * L, L)] = gbuf[e, pl.ds(r * L, L)] * wi
            return carry

        lax.fori_loop(0, CHUNK // L, _group, 0)

        pltpu.sync_copy(gbuf, acc.at[rows_v.at[j]], add=True)
        return carry

    lax.fori_loop(0, NCHUNK, _chunk, 0)

    plsc.subcore_barrier()
    pltpu.sync_copy(acc.at[pl.ds(base, ROWS_PER_TILE)],
                    partial_h.at[cid, pl.ds(base, ROWS_PER_TILE)])

    @pl.when(sid == NS - 1)
    def _flush_tail():
        pltpu.sync_copy(
            acc.at[pl.ds(NS * ROWS_PER_TILE, ROWS_REM)],
            partial_h.at[cid, pl.ds(NS * ROWS_PER_TILE, ROWS_REM)])


def _combine_body(p_ref, o_ref):
    o_ref[...] = p_ref[0] + p_ref[1]


_ROWS_BLK = 1000


@jax.jit
def _combine(partial):
    return pl.pallas_call(
        _combine_body,
        out_shape=jax.ShapeDtypeStruct((N_NODES, D_FEAT), jnp.float32),
        grid=(N_NODES // _ROWS_BLK,),
        in_specs=[pl.BlockSpec((NC, _ROWS_BLK, D_FEAT), lambda i: (0, i, 0))],
        out_specs=pl.BlockSpec((_ROWS_BLK, D_FEAT), lambda i: (i, 0)),
    )(partial)


@jax.jit
def kernel(edge_index, edge_weight, feats):
    rows = edge_index[0].astype(jnp.int32)
    cols = edge_index[1].astype(jnp.int32)
    w = edge_weight.astype(jnp.float32)

    pad = E_PAD - N_EDGES
    rows = jnp.concatenate([rows, jnp.zeros((pad,), jnp.int32)])
    cols = jnp.concatenate([cols, jnp.zeros((pad,), jnp.int32)])
    w = jnp.concatenate([w, jnp.zeros((pad,), jnp.float32)])

    rows = rows.reshape(NW, NCHUNK, CHUNK)
    cols = cols.reshape(NW, NCHUNK, CHUNK)
    w = w.reshape(NW, NCHUNK * CHUNK)

    partial = _sc_aggregate(cols, rows, w, feats)
    return _combine(partial)
